# Initial kernel scaffold; baseline (speedup 1.0000x reference)
import jax
import jax.numpy as jnp
from jax.experimental import pallas as pl


def kernel(h, adj, W, a):
    raise NotImplementedError("write your pallas kernel here")


# speedup vs baseline: 20.6778x; 20.6778x over previous
import dataclasses
import functools

import jax
import jax.numpy as jnp
from jax import lax
from jax.experimental import pallas as pl
from jax.experimental.pallas import tpu as pltpu
from jax.experimental.pallas import tpu_sc as plsc

N = 10000
E = 320000
IN_F = 128
OUT_F = 32
HEADS = 8
ALPHA = 0.2

NP = 10240
DUMMY = N
C = 512
NSUB = 16
KPT = 40
NCHUNK = NSUB * KPT
E_PAD = NCHUNK * C
BR = 1280
NBR = NP // BR


def _tc_proj_kernel(h_ref, wc_ref, a1_ref, a2_ref, wh_ref, ti_ref, tj_ref):
    q = pl.program_id(1)
    whb = jnp.dot(h_ref[...], wc_ref[0],
                  preferred_element_type=jnp.float32,
                  precision=lax.Precision.HIGHEST)
    wh_ref[...] = whb
    tip = jnp.dot(whb, a1_ref[0],
                  preferred_element_type=jnp.float32,
                  precision=lax.Precision.HIGHEST)
    tjp = jnp.dot(whb, a2_ref[0],
                  preferred_element_type=jnp.float32,
                  precision=lax.Precision.HIGHEST)

    @pl.when(q % 2 == 0)
    def _():
        ti_ref[...] = tip
        tj_ref[...] = tjp

    @pl.when(q % 2 == 1)
    def _():
        ti_ref[...] += tip
        tj_ref[...] += tjp


def _tc_proj(h_pad, wc, a1q, a2q):
    return pl.pallas_call(
        _tc_proj_kernel,
        grid=(NBR, 4),
        in_specs=[
            pl.BlockSpec((BR, IN_F), lambda i, q: (i, 0)),
            pl.BlockSpec((1, IN_F, 64), lambda i, q: (q, 0, 0)),
            pl.BlockSpec((1, 64, 16), lambda i, q: (q, 0, 0)),
            pl.BlockSpec((1, 64, 16), lambda i, q: (q, 0, 0)),
        ],
        out_specs=[
            pl.BlockSpec((BR, 64), lambda i, q: (q * NBR + i, 0)),
            pl.BlockSpec((BR, 16), lambda i, q: ((q // 2) * NBR + i, 0)),
            pl.BlockSpec((BR, 16), lambda i, q: ((q // 2) * NBR + i, 0)),
        ],
        out_shape=[
            jax.ShapeDtypeStruct((4 * NP, 64), jnp.float32),
            jax.ShapeDtypeStruct((2 * NP, 16), jnp.float32),
            jax.ShapeDtypeStruct((2 * NP, 16), jnp.float32),
        ],
    )(h_pad, wc, a1q, a2q)


def _sc_gat(row3, col3, ti, tj, wh4):
    mesh = plsc.VectorSubcoreMesh(core_axis_name="c", subcore_axis_name="s")
    cp = pltpu.CompilerParams()
    if "needs_layout_passes" in pltpu.CompilerParams.__dataclass_fields__:
        cp = dataclasses.replace(cp, needs_layout_passes=False)
    if "use_tc_tiling_on_sc" in pltpu.CompilerParams.__dataclass_fields__:
        cp = dataclasses.replace(cp, use_tc_tiling_on_sc=False)

    @functools.partial(
        pl.kernel,
        compiler_params=cp,
        out_type=(
            jax.ShapeDtypeStruct((4 * NP, 64), jnp.float32),
            jax.ShapeDtypeStruct((2 * NCHUNK, C, 16), jnp.float32),
            jax.ShapeDtypeStruct((2 * NCHUNK, C, 16), jnp.float32),
        ),
        mesh=mesh,
        scratch_types=[
            pltpu.VMEM((4, 128), jnp.int32),
            pltpu.VMEM((4, 128), jnp.int32),
            pltpu.VMEM((4, 128), jnp.int32),
            pltpu.VMEM((C, 16), jnp.float32),
            pltpu.VMEM((C, 16), jnp.float32),
            pltpu.VMEM((C, 16), jnp.float32),
            pltpu.VMEM((C, 16), jnp.float32),
            pltpu.VMEM((C, 64), jnp.float32),
            pltpu.VMEM_SHARED((NP, 16), jnp.float32),
            pltpu.VMEM_SHARED((NP, 64), jnp.float32),
        ],
    )
    def k(row3_r, col3_r, ti_r, tj_r, wh4_r, out_r, e_scr, ee_scr,
          rowv, colv, adjv, buf_a, buf_b, buf_e, buf_s, rowsb, S12, OSP):
        c = lax.axis_index("c")
        s = lax.axis_index("s")

        zero16 = jnp.zeros((16,), jnp.float32)
        iota16 = lax.iota(jnp.int32, 16)
        mask48 = (iota16 >= 4) & (iota16 < 8)
        shidx = jnp.maximum(iota16 - 4, 0)
        nz = NP // NSUB

        def zero_rowsb():
            @pl.loop(0, C)
            def _(r):
                for t in range(4):
                    rowsb[r, pl.ds(t * 16, 16)] = zero16

        def zero_osp():
            pltpu.sync_copy(rowsb, OSP.at[pl.ds(s * nz, C)])
            pltpu.sync_copy(rowsb.at[pl.ds(0, nz - C)],
                            OSP.at[pl.ds(s * nz + C, nz - C)])

        zero_rowsb()

        @pl.loop(0, C)
        def _(r):
            buf_e[r, :] = zero16

        zero_osp()
        pltpu.sync_copy(buf_e, S12.at[pl.ds(s * nz, C)])
        pltpu.sync_copy(buf_e.at[pl.ds(0, nz - C)],
                        S12.at[pl.ds(s * nz + C, nz - C)])
        plsc.subcore_barrier()

        def adjust(src, base):
            for j in range(4):
                @pl.loop(0, 8)
                def _(qq, j=j):
                    sl = pl.ds(qq * 16, 16)
                    adjv[j, sl] = src[j, sl] + base

        @pl.loop(0, KPT)
        def _(kk):
            cid = s * KPT + kk
            gcid = c * NCHUNK + cid
            pltpu.sync_copy(row3_r.at[cid], rowv)
            pltpu.sync_copy(col3_r.at[cid], colv)
            adjust(rowv, c * NP)
            for j in range(4):
                pltpu.sync_copy(ti_r.at[adjv.at[j]],
                                buf_a.at[pl.ds(j * 128, 128)])
            adjust(colv, c * NP)
            for j in range(4):
                pltpu.sync_copy(tj_r.at[adjv.at[j]],
                                buf_b.at[pl.ds(j * 128, 128)])

            @pl.loop(0, C)
            def _(r):
                v = buf_a[r, :] + buf_b[r, :]
                buf_e[r, :] = jnp.where(v > 0, v, v * ALPHA)

            pltpu.sync_copy(buf_e, e_scr.at[gcid])
            for j in range(4):
                pltpu.sync_copy(buf_e.at[pl.ds(j * 128, 128)],
                                S12.at[rowv.at[j]], add=True)
        plsc.subcore_barrier()

        @pl.loop(0, KPT)
        def _(kk):
            cid = s * KPT + kk
            gcid = c * NCHUNK + cid
            pltpu.sync_copy(row3_r.at[cid], rowv)
            pltpu.sync_copy(e_scr.at[gcid], buf_e)
            for j in range(4):
                pltpu.sync_copy(S12.at[rowv.at[j]],
                                buf_s.at[pl.ds(j * 128, 128)])

            @pl.loop(0, C)
            def _(r):
                ex = jnp.exp(buf_e[r, :] - buf_s[r, :])
                sh = ex.at[shidx].get(mode=lax.GatherScatterMode.PROMISE_IN_BOUNDS)
                buf_b[r, :] = jnp.where(mask48, sh, 0.0)

            pltpu.sync_copy(buf_b, ee_scr.at[gcid])
            for j in range(4):
                pltpu.sync_copy(buf_b.at[pl.ds(j * 128, 128)],
                                S12.at[rowv.at[j]], add=True)
        plsc.subcore_barrier()

        for half in range(2):
            @pl.loop(0, KPT)
            def _(kk, half=half):
                cid = s * KPT + kk
                gcid = c * NCHUNK + cid
                pltpu.sync_copy(row3_r.at[cid], rowv)
                pltpu.sync_copy(col3_r.at[cid], colv)
                pltpu.sync_copy(ee_scr.at[gcid], buf_b)
                for j in range(4):
                    pltpu.sync_copy(S12.at[rowv.at[j]],
                                    buf_s.at[pl.ds(j * 128, 128)])

                @pl.loop(0, C)
                def _(r):
                    buf_b[r, :] = buf_b[r, :] / (buf_s[r, :] + 1e-16)

                adjust(colv, (c * 2 + half) * NP)
                for j in range(4):
                    pltpu.sync_copy(wh4_r.at[adjv.at[j]],
                                    rowsb.at[pl.ds(j * 128, 128)])

                @pl.loop(0, C)
                def _(ce):
                    for hh in range(2):
                        av = plsc.load_gather(
                            buf_b,
                            [jnp.full((16,), ce, jnp.int32),
                             jnp.full((16,), 4 + half * 2 + hh, jnp.int32)])
                        for t in range(2):
                            sl = pl.ds(hh * 32 + t * 16, 16)
                            rowsb[ce, sl] = rowsb[ce, sl] * av
                for j in range(4):
                    pltpu.sync_copy(rowsb.at[pl.ds(j * 128, 128)],
                                    OSP.at[rowv.at[j]], add=True)
            plsc.subcore_barrier()

            pltpu.sync_copy(
                OSP.at[pl.ds(s * nz, nz)],
                out_r.at[pl.ds((c * 2 + half) * NP + s * nz, nz)])
            if half == 0:
                plsc.subcore_barrier()
                zero_rowsb()
                zero_osp()
                plsc.subcore_barrier()

    return k(row3, col3, ti, tj, wh4)


def kernel(h, adj, W, a):
    h = h.astype(jnp.float32)
    W = W.astype(jnp.float32)
    a = a.astype(jnp.float32)

    h_pad = jnp.zeros((NP, IN_F), jnp.float32).at[:N].set(h)
    wc = (jnp.transpose(W, (1, 0, 2)).reshape(IN_F, 4, 64)
          .transpose(1, 0, 2))
    a1 = a[:, :OUT_F, 0].reshape(2, 4, OUT_F)
    a2 = a[:, OUT_F:, 0].reshape(2, 4, OUT_F)
    eye = jnp.eye(4, 16, dtype=jnp.float32)
    a1b = (a1[:, :, :, None] * eye[None, :, None, :]).reshape(2, 128, 16)
    a2b = (a2[:, :, :, None] * eye[None, :, None, :]).reshape(2, 128, 16)
    a1q = a1b.reshape(4, 64, 16)
    a2q = a2b.reshape(4, 64, 16)

    row = adj[0].astype(jnp.int32)
    col = adj[1].astype(jnp.int32)
    rowp = jnp.full((E_PAD,), DUMMY, jnp.int32).at[:E].set(row)
    colp = jnp.full((E_PAD,), DUMMY, jnp.int32).at[:E].set(col)
    row3 = rowp.reshape(NCHUNK, 4, 128)
    col3 = colp.reshape(NCHUNK, 4, 128)

    wh4, ti, tj = _tc_proj(h_pad, wc, a1q, a2q)
    out4, _, _ = _sc_gat(row3, col3, ti, tj, wh4)
    return jnp.concatenate(
        [out4[q * NP:q * NP + N] for q in range(4)], axis=1)


# speedup vs baseline: 26.8077x; 1.2965x over previous
import dataclasses
import functools

import jax
import jax.numpy as jnp
from jax import lax
from jax.experimental import pallas as pl
from jax.experimental.pallas import tpu as pltpu
from jax.experimental.pallas import tpu_sc as plsc

N = 10000
E = 320000
IN_F = 128
OUT_F = 32
HEADS = 8
ALPHA = 0.2

NP = 10240
DUMMY = N
C = 512
NSUB = 16
KPT = 40
NCHUNK = NSUB * KPT
E_PAD = NCHUNK * C
BR = 1280
NBR = NP // BR


def _tc_proj_kernel(h_ref, wc_ref, a1_ref, a2_ref, wh_ref, ti_ref, tj_ref):
    q = pl.program_id(1)
    whb = jnp.dot(h_ref[...], wc_ref[0],
                  preferred_element_type=jnp.float32,
                  precision=lax.Precision.HIGHEST)
    wh_ref[...] = whb
    tip = jnp.dot(whb, a1_ref[0],
                  preferred_element_type=jnp.float32,
                  precision=lax.Precision.HIGHEST)
    tjp = jnp.dot(whb, a2_ref[0],
                  preferred_element_type=jnp.float32,
                  precision=lax.Precision.HIGHEST)

    @pl.when(q % 2 == 0)
    def _():
        ti_ref[...] = tip
        tj_ref[...] = tjp

    @pl.when(q % 2 == 1)
    def _():
        ti_ref[...] += tip
        tj_ref[...] += tjp


def _tc_proj(h_pad, wc, a1q, a2q):
    return pl.pallas_call(
        _tc_proj_kernel,
        grid=(NBR, 4),
        in_specs=[
            pl.BlockSpec((BR, IN_F), lambda i, q: (i, 0)),
            pl.BlockSpec((1, IN_F, 64), lambda i, q: (q, 0, 0)),
            pl.BlockSpec((1, 64, 16), lambda i, q: (q, 0, 0)),
            pl.BlockSpec((1, 64, 16), lambda i, q: (q, 0, 0)),
        ],
        out_specs=[
            pl.BlockSpec((BR, 64), lambda i, q: (q * NBR + i, 0)),
            pl.BlockSpec((BR, 16), lambda i, q: ((q // 2) * NBR + i, 0)),
            pl.BlockSpec((BR, 16), lambda i, q: ((q // 2) * NBR + i, 0)),
        ],
        out_shape=[
            jax.ShapeDtypeStruct((4 * NP, 64), jnp.float32),
            jax.ShapeDtypeStruct((2 * NP, 16), jnp.float32),
            jax.ShapeDtypeStruct((2 * NP, 16), jnp.float32),
        ],
    )(h_pad, wc, a1q, a2q)


def _sc_gat(row3, col3, ti, tj, wh4):
    mesh = plsc.VectorSubcoreMesh(core_axis_name="c", subcore_axis_name="s")
    cp = pltpu.CompilerParams()
    if "needs_layout_passes" in pltpu.CompilerParams.__dataclass_fields__:
        cp = dataclasses.replace(cp, needs_layout_passes=False)
    if "use_tc_tiling_on_sc" in pltpu.CompilerParams.__dataclass_fields__:
        cp = dataclasses.replace(cp, use_tc_tiling_on_sc=False)

    @functools.partial(
        pl.kernel,
        compiler_params=cp,
        out_type=(
            jax.ShapeDtypeStruct((4 * NP, 64), jnp.float32),
            jax.ShapeDtypeStruct((2 * NCHUNK, C, 16), jnp.float32),
            jax.ShapeDtypeStruct((2 * NCHUNK, C, 16), jnp.float32),
        ),
        mesh=mesh,
        scratch_types=[
            pltpu.VMEM((4, 128), jnp.int32),
            pltpu.VMEM((4, 128), jnp.int32),
            pltpu.VMEM((4, 128), jnp.int32),
            pltpu.VMEM((C, 16), jnp.float32),
            pltpu.VMEM((C, 16), jnp.float32),
            pltpu.VMEM((C, 16), jnp.float32),
            pltpu.VMEM((C, 16), jnp.float32),
            pltpu.VMEM((C, 64), jnp.float32),
            pltpu.VMEM_SHARED((NP, 16), jnp.float32),
            pltpu.VMEM_SHARED((NP, 64), jnp.float32),
        ],
    )
    def k(row3_r, col3_r, ti_r, tj_r, wh4_r, out_r, e_scr, ee_scr,
          rowv, colv, adjv, buf_a, buf_b, buf_e, buf_s, rowsb, S12, OSP):
        c = lax.axis_index("c")
        s = lax.axis_index("s")

        zero16 = jnp.zeros((16,), jnp.float32)
        iota16 = lax.iota(jnp.int32, 16)
        mask48 = (iota16 >= 4) & (iota16 < 8)
        shidx = jnp.maximum(iota16 - 4, 0)
        nz = NP // NSUB

        def zero_rowsb():
            @plsc.parallel_loop(0, C, unroll=4)
            def _(r):
                for t in range(4):
                    rowsb[r, pl.ds(t * 16, 16)] = zero16

        def zero_osp():
            pltpu.sync_copy(rowsb, OSP.at[pl.ds(s * nz, C)])
            pltpu.sync_copy(rowsb.at[pl.ds(0, nz - C)],
                            OSP.at[pl.ds(s * nz + C, nz - C)])

        zero_rowsb()

        @plsc.parallel_loop(0, C, unroll=4)
        def _(r):
            buf_e[r, :] = zero16

        zero_osp()
        pltpu.sync_copy(buf_e, S12.at[pl.ds(s * nz, C)])
        pltpu.sync_copy(buf_e.at[pl.ds(0, nz - C)],
                        S12.at[pl.ds(s * nz + C, nz - C)])
        plsc.subcore_barrier()

        def adjust(src, base):
            for j in range(4):
                @plsc.parallel_loop(0, 8, unroll=4)
                def _(qq, j=j):
                    sl = pl.ds(qq * 16, 16)
                    adjv[j, sl] = src[j, sl] + base

        @pl.loop(0, KPT)
        def _(kk):
            cid = s * KPT + kk
            gcid = c * NCHUNK + cid
            pltpu.sync_copy(row3_r.at[cid], rowv)
            pltpu.sync_copy(col3_r.at[cid], colv)
            adjust(rowv, c * NP)
            for j in range(4):
                pltpu.sync_copy(ti_r.at[adjv.at[j]],
                                buf_a.at[pl.ds(j * 128, 128)])
            adjust(colv, c * NP)
            for j in range(4):
                pltpu.sync_copy(tj_r.at[adjv.at[j]],
                                buf_b.at[pl.ds(j * 128, 128)])

            @plsc.parallel_loop(0, C, unroll=4)
            def _(r):
                v = buf_a[r, :] + buf_b[r, :]
                buf_e[r, :] = jnp.where(v > 0, v, v * ALPHA)

            pltpu.sync_copy(buf_e, e_scr.at[gcid])
            for j in range(4):
                pltpu.sync_copy(buf_e.at[pl.ds(j * 128, 128)],
                                S12.at[rowv.at[j]], add=True)
        plsc.subcore_barrier()

        @pl.loop(0, KPT)
        def _(kk):
            cid = s * KPT + kk
            gcid = c * NCHUNK + cid
            pltpu.sync_copy(row3_r.at[cid], rowv)
            pltpu.sync_copy(e_scr.at[gcid], buf_e)
            for j in range(4):
                pltpu.sync_copy(S12.at[rowv.at[j]],
                                buf_s.at[pl.ds(j * 128, 128)])

            @plsc.parallel_loop(0, C, unroll=4)
            def _(r):
                ex = jnp.exp(buf_e[r, :] - buf_s[r, :])
                sh = ex.at[shidx].get(mode=lax.GatherScatterMode.PROMISE_IN_BOUNDS)
                buf_b[r, :] = jnp.where(mask48, sh, 0.0)

            pltpu.sync_copy(buf_b, ee_scr.at[gcid])
            for j in range(4):
                pltpu.sync_copy(buf_b.at[pl.ds(j * 128, 128)],
                                S12.at[rowv.at[j]], add=True)
        plsc.subcore_barrier()

        for half in range(2):
            @pl.loop(0, KPT)
            def _(kk, half=half):
                cid = s * KPT + kk
                gcid = c * NCHUNK + cid
                pltpu.sync_copy(row3_r.at[cid], rowv)
                pltpu.sync_copy(col3_r.at[cid], colv)
                pltpu.sync_copy(ee_scr.at[gcid], buf_b)
                for j in range(4):
                    pltpu.sync_copy(S12.at[rowv.at[j]],
                                    buf_s.at[pl.ds(j * 128, 128)])

                @plsc.parallel_loop(0, C, unroll=4)
                def _(r):
                    buf_b[r, :] = buf_b[r, :] / (buf_s[r, :] + 1e-16)

                adjust(colv, (c * 2 + half) * NP)
                for j in range(4):
                    pltpu.sync_copy(wh4_r.at[adjv.at[j]],
                                    rowsb.at[pl.ds(j * 128, 128)])

                @plsc.parallel_loop(0, C, unroll=2)
                def _(ce):
                    for hh in range(2):
                        av = plsc.load_gather(
                            buf_b,
                            [jnp.full((16,), ce, jnp.int32),
                             jnp.full((16,), 4 + half * 2 + hh, jnp.int32)])
                        for t in range(2):
                            sl = pl.ds(hh * 32 + t * 16, 16)
                            rowsb[ce, sl] = rowsb[ce, sl] * av
                for j in range(4):
                    pltpu.sync_copy(rowsb.at[pl.ds(j * 128, 128)],
                                    OSP.at[rowv.at[j]], add=True)
            plsc.subcore_barrier()

            pltpu.sync_copy(
                OSP.at[pl.ds(s * nz, nz)],
                out_r.at[pl.ds((c * 2 + half) * NP + s * nz, nz)])
            if half == 0:
                plsc.subcore_barrier()
                zero_rowsb()
                zero_osp()
                plsc.subcore_barrier()

    return k(row3, col3, ti, tj, wh4)


def kernel(h, adj, W, a):
    h = h.astype(jnp.float32)
    W = W.astype(jnp.float32)
    a = a.astype(jnp.float32)

    h_pad = jnp.zeros((NP, IN_F), jnp.float32).at[:N].set(h)
    wc = (jnp.transpose(W, (1, 0, 2)).reshape(IN_F, 4, 64)
          .transpose(1, 0, 2))
    a1 = a[:, :OUT_F, 0].reshape(2, 4, OUT_F)
    a2 = a[:, OUT_F:, 0].reshape(2, 4, OUT_F)
    eye = jnp.eye(4, 16, dtype=jnp.float32)
    a1b = (a1[:, :, :, None] * eye[None, :, None, :]).reshape(2, 128, 16)
    a2b = (a2[:, :, :, None] * eye[None, :, None, :]).reshape(2, 128, 16)
    a1q = a1b.reshape(4, 64, 16)
    a2q = a2b.reshape(4, 64, 16)

    row = adj[0].astype(jnp.int32)
    col = adj[1].astype(jnp.int32)
    rowp = jnp.full((E_PAD,), DUMMY, jnp.int32).at[:E].set(row)
    colp = jnp.full((E_PAD,), DUMMY, jnp.int32).at[:E].set(col)
    row3 = rowp.reshape(NCHUNK, 4, 128)
    col3 = colp.reshape(NCHUNK, 4, 128)

    wh4, ti, tj = _tc_proj(h_pad, wc, a1q, a2q)
    out4, _, _ = _sc_gat(row3, col3, ti, tj, wh4)
    return jnp.concatenate(
        [out4[q * NP:q * NP + N] for q in range(4)], axis=1)


# speedup vs baseline: 31.5883x; 1.1783x over previous
import dataclasses
import functools

import jax
import jax.numpy as jnp
from jax import lax
from jax.experimental import pallas as pl
from jax.experimental.pallas import tpu as pltpu
from jax.experimental.pallas import tpu_sc as plsc

N = 10000
E = 320000
IN_F = 128
OUT_F = 32
HEADS = 8
ALPHA = 0.2

NP = 10240
DUMMY = N
C = 512
NSUB = 16
KPT = 40
NCHUNK = NSUB * KPT
E_PAD = NCHUNK * C
BR = 1280
NBR = NP // BR


def _tc_proj_kernel(h_ref, wc_ref, a1_ref, a2_ref, wh_ref, ti_ref, tj_ref):
    q = pl.program_id(1)
    whb = jnp.dot(h_ref[...], wc_ref[0],
                  preferred_element_type=jnp.float32,
                  precision=lax.Precision.HIGHEST)
    wh_ref[...] = whb
    tip = jnp.dot(whb, a1_ref[0],
                  preferred_element_type=jnp.float32,
                  precision=lax.Precision.HIGHEST)
    tjp = jnp.dot(whb, a2_ref[0],
                  preferred_element_type=jnp.float32,
                  precision=lax.Precision.HIGHEST)

    @pl.when(q % 2 == 0)
    def _():
        ti_ref[...] = tip
        tj_ref[...] = tjp

    @pl.when(q % 2 == 1)
    def _():
        ti_ref[...] += tip
        tj_ref[...] += tjp


def _tc_proj(h_pad, wc, a1q, a2q):
    return pl.pallas_call(
        _tc_proj_kernel,
        grid=(NBR, 4),
        in_specs=[
            pl.BlockSpec((BR, IN_F), lambda i, q: (i, 0)),
            pl.BlockSpec((1, IN_F, 64), lambda i, q: (q, 0, 0)),
            pl.BlockSpec((1, 64, 16), lambda i, q: (q, 0, 0)),
            pl.BlockSpec((1, 64, 16), lambda i, q: (q, 0, 0)),
        ],
        out_specs=[
            pl.BlockSpec((BR, 64), lambda i, q: (q * NBR + i, 0)),
            pl.BlockSpec((BR, 16), lambda i, q: ((q // 2) * NBR + i, 0)),
            pl.BlockSpec((BR, 16), lambda i, q: ((q // 2) * NBR + i, 0)),
        ],
        out_shape=[
            jax.ShapeDtypeStruct((4 * NP, 64), jnp.float32),
            jax.ShapeDtypeStruct((2 * NP, 16), jnp.float32),
            jax.ShapeDtypeStruct((2 * NP, 16), jnp.float32),
        ],
    )(h_pad, wc, a1q, a2q)


def _sc_gat(row3, col3, ti, tj, wh4):
    mesh = plsc.VectorSubcoreMesh(core_axis_name="c", subcore_axis_name="s")
    cp = pltpu.CompilerParams()
    if "needs_layout_passes" in pltpu.CompilerParams.__dataclass_fields__:
        cp = dataclasses.replace(cp, needs_layout_passes=False)
    if "use_tc_tiling_on_sc" in pltpu.CompilerParams.__dataclass_fields__:
        cp = dataclasses.replace(cp, use_tc_tiling_on_sc=False)

    @functools.partial(
        pl.kernel,
        compiler_params=cp,
        out_type=(
            jax.ShapeDtypeStruct((4 * NP, 64), jnp.float32),
            jax.ShapeDtypeStruct((2 * NCHUNK, C, 16), jnp.float32),
            jax.ShapeDtypeStruct((2 * NCHUNK, C, 16), jnp.float32),
        ),
        mesh=mesh,
        scratch_types=[
            pltpu.VMEM((C,), jnp.int32),
            pltpu.VMEM((C,), jnp.int32),
            pltpu.VMEM((C,), jnp.int32),
            pltpu.VMEM((C, 16), jnp.float32),
            pltpu.VMEM((C, 16), jnp.float32),
            pltpu.VMEM((C, 16), jnp.float32),
            pltpu.VMEM((C, 16), jnp.float32),
            pltpu.VMEM((C, 64), jnp.float32),
            pltpu.VMEM((C,), jnp.int32),
            pltpu.VMEM_SHARED((NP, 16), jnp.float32),
            pltpu.VMEM_SHARED((NP, 64), jnp.float32),
        ],
    )
    def k(row3_r, col3_r, ti_r, tj_r, wh4_r, out_r, e_scr, ee_scr,
          rowv, colv, adjv, buf_a, buf_b, buf_e, buf_s, rowsb, adjc,
          S12, OSP):
        c = lax.axis_index("c")
        s = lax.axis_index("s")

        zero16 = jnp.zeros((16,), jnp.float32)
        iota16 = lax.iota(jnp.int32, 16)
        mask48 = (iota16 >= 4) & (iota16 < 8)
        shidx = jnp.maximum(iota16 - 4, 0)
        nz = NP // NSUB

        def zero_rowsb():
            @plsc.parallel_loop(0, C, unroll=4)
            def _(r):
                for t in range(4):
                    rowsb[r, pl.ds(t * 16, 16)] = zero16

        def zero_osp():
            pltpu.sync_copy(rowsb, OSP.at[pl.ds(s * nz, C)])
            pltpu.sync_copy(rowsb.at[pl.ds(0, nz - C)],
                            OSP.at[pl.ds(s * nz + C, nz - C)])

        zero_rowsb()

        @plsc.parallel_loop(0, C, unroll=4)
        def _(r):
            buf_e[r, :] = zero16

        zero_osp()
        pltpu.sync_copy(buf_e, S12.at[pl.ds(s * nz, C)])
        pltpu.sync_copy(buf_e.at[pl.ds(0, nz - C)],
                        S12.at[pl.ds(s * nz + C, nz - C)])
        plsc.subcore_barrier()

        def adjust(src, dst, base):
            @plsc.parallel_loop(0, C // 16, unroll=4)
            def _(qq):
                sl = pl.ds(qq * 16, 16)
                dst[sl] = src[sl] + base

        @pl.loop(0, KPT)
        def _(kk):
            cid = s * KPT + kk
            gcid = c * NCHUNK + cid
            pltpu.sync_copy(row3_r.at[cid], rowv)
            pltpu.sync_copy(col3_r.at[cid], colv)
            adjust(rowv, adjv, c * NP)
            adjust(colv, adjc, c * NP)
            pltpu.sync_copy(ti_r.at[adjv], buf_a)
            pltpu.sync_copy(tj_r.at[adjc], buf_b)

            @plsc.parallel_loop(0, C, unroll=4)
            def _(r):
                v = buf_a[r, :] + buf_b[r, :]
                buf_e[r, :] = jnp.where(v > 0, v, v * ALPHA)

            pltpu.sync_copy(buf_e, e_scr.at[gcid])
            pltpu.sync_copy(buf_e, S12.at[rowv], add=True)
        plsc.subcore_barrier()

        @pl.loop(0, KPT)
        def _(kk):
            cid = s * KPT + kk
            gcid = c * NCHUNK + cid
            pltpu.sync_copy(row3_r.at[cid], rowv)
            pltpu.sync_copy(e_scr.at[gcid], buf_e)
            pltpu.sync_copy(S12.at[rowv], buf_s)

            @plsc.parallel_loop(0, C, unroll=4)
            def _(r):
                ex = jnp.exp(buf_e[r, :] - buf_s[r, :])
                sh = ex.at[shidx].get(mode=lax.GatherScatterMode.PROMISE_IN_BOUNDS)
                buf_b[r, :] = jnp.where(mask48, sh, 0.0)

            pltpu.sync_copy(buf_b, ee_scr.at[gcid])
            pltpu.sync_copy(buf_b, S12.at[rowv], add=True)
        plsc.subcore_barrier()

        for half in range(2):
            @pl.loop(0, KPT)
            def _(kk, half=half):
                cid = s * KPT + kk
                gcid = c * NCHUNK + cid
                pltpu.sync_copy(row3_r.at[cid], rowv)
                pltpu.sync_copy(col3_r.at[cid], colv)
                pltpu.sync_copy(ee_scr.at[gcid], buf_b)
                pltpu.sync_copy(S12.at[rowv], buf_s)

                @plsc.parallel_loop(0, C, unroll=4)
                def _(r):
                    buf_b[r, :] = buf_b[r, :] / (buf_s[r, :] + 1e-16)

                adjust(colv, adjc, (c * 2 + half) * NP)
                pltpu.sync_copy(wh4_r.at[adjc], rowsb)

                @plsc.parallel_loop(0, C, unroll=2)
                def _(ce):
                    for hh in range(2):
                        av = plsc.load_gather(
                            buf_b,
                            [jnp.full((16,), ce, jnp.int32),
                             jnp.full((16,), 4 + half * 2 + hh, jnp.int32)])
                        for t in range(2):
                            sl = pl.ds(hh * 32 + t * 16, 16)
                            rowsb[ce, sl] = rowsb[ce, sl] * av
                pltpu.sync_copy(rowsb, OSP.at[rowv], add=True)
            plsc.subcore_barrier()

            pltpu.sync_copy(
                OSP.at[pl.ds(s * nz, nz)],
                out_r.at[pl.ds((c * 2 + half) * NP + s * nz, nz)])
            if half == 0:
                plsc.subcore_barrier()
                zero_rowsb()
                zero_osp()
                plsc.subcore_barrier()

    return k(row3, col3, ti, tj, wh4)


def kernel(h, adj, W, a):
    h = h.astype(jnp.float32)
    W = W.astype(jnp.float32)
    a = a.astype(jnp.float32)

    h_pad = jnp.zeros((NP, IN_F), jnp.float32).at[:N].set(h)
    wc = (jnp.transpose(W, (1, 0, 2)).reshape(IN_F, 4, 64)
          .transpose(1, 0, 2))
    a1 = a[:, :OUT_F, 0].reshape(2, 4, OUT_F)
    a2 = a[:, OUT_F:, 0].reshape(2, 4, OUT_F)
    eye = jnp.eye(4, 16, dtype=jnp.float32)
    a1b = (a1[:, :, :, None] * eye[None, :, None, :]).reshape(2, 128, 16)
    a2b = (a2[:, :, :, None] * eye[None, :, None, :]).reshape(2, 128, 16)
    a1q = a1b.reshape(4, 64, 16)
    a2q = a2b.reshape(4, 64, 16)

    row = adj[0].astype(jnp.int32)
    col = adj[1].astype(jnp.int32)
    rowp = jnp.full((E_PAD,), DUMMY, jnp.int32).at[:E].set(row)
    colp = jnp.full((E_PAD,), DUMMY, jnp.int32).at[:E].set(col)
    row3 = rowp.reshape(NCHUNK, C)
    col3 = colp.reshape(NCHUNK, C)

    wh4, ti, tj = _tc_proj(h_pad, wc, a1q, a2q)
    out4, _, _ = _sc_gat(row3, col3, ti, tj, wh4)
    return jnp.concatenate(
        [out4[q * NP:q * NP + N] for q in range(4)], axis=1)


# speedup vs baseline: 32.9936x; 1.0445x over previous
import dataclasses
import functools

import jax
import jax.numpy as jnp
from jax import lax
from jax.experimental import pallas as pl
from jax.experimental.pallas import tpu as pltpu
from jax.experimental.pallas import tpu_sc as plsc

N = 10000
E = 320000
IN_F = 128
OUT_F = 32
HEADS = 8
ALPHA = 0.2

NP = 10240
DUMMY = N
C = 512
NSUB = 16
KPT = 40
NCHUNK = NSUB * KPT
E_PAD = NCHUNK * C
BR = 1280
NBR = NP // BR


def _tc_proj_kernel(h_ref, wc_ref, a1_ref, a2_ref, wh_ref, ti_ref, tj_ref):
    q = pl.program_id(1)
    whb = jnp.dot(h_ref[...], wc_ref[0],
                  preferred_element_type=jnp.float32,
                  precision=lax.Precision.HIGHEST)
    wh_ref[...] = whb
    tip = jnp.dot(whb, a1_ref[0],
                  preferred_element_type=jnp.float32,
                  precision=lax.Precision.HIGHEST)
    tjp = jnp.dot(whb, a2_ref[0],
                  preferred_element_type=jnp.float32,
                  precision=lax.Precision.HIGHEST)

    @pl.when(q % 2 == 0)
    def _():
        ti_ref[...] = tip
        tj_ref[...] = tjp

    @pl.when(q % 2 == 1)
    def _():
        ti_ref[...] += tip
        tj_ref[...] += tjp


def _tc_proj(h_pad, wc, a1q, a2q):
    return pl.pallas_call(
        _tc_proj_kernel,
        grid=(NBR, 4),
        in_specs=[
            pl.BlockSpec((BR, IN_F), lambda i, q: (i, 0)),
            pl.BlockSpec((1, IN_F, 64), lambda i, q: (q, 0, 0)),
            pl.BlockSpec((1, 64, 16), lambda i, q: (q, 0, 0)),
            pl.BlockSpec((1, 64, 16), lambda i, q: (q, 0, 0)),
        ],
        out_specs=[
            pl.BlockSpec((BR, 64), lambda i, q: (q * NBR + i, 0)),
            pl.BlockSpec((BR, 16), lambda i, q: ((q // 2) * NBR + i, 0)),
            pl.BlockSpec((BR, 16), lambda i, q: ((q // 2) * NBR + i, 0)),
        ],
        out_shape=[
            jax.ShapeDtypeStruct((4 * NP, 64), jnp.float32),
            jax.ShapeDtypeStruct((2 * NP, 16), jnp.float32),
            jax.ShapeDtypeStruct((2 * NP, 16), jnp.float32),
        ],
    )(h_pad, wc, a1q, a2q)


def _sc_gat(rc2, ti, tj, wh4):
    mesh = plsc.VectorSubcoreMesh(core_axis_name="c", subcore_axis_name="s")
    cp = pltpu.CompilerParams()
    if "needs_layout_passes" in pltpu.CompilerParams.__dataclass_fields__:
        cp = dataclasses.replace(cp, needs_layout_passes=False)
    if "use_tc_tiling_on_sc" in pltpu.CompilerParams.__dataclass_fields__:
        cp = dataclasses.replace(cp, use_tc_tiling_on_sc=False)

    @functools.partial(
        pl.kernel,
        compiler_params=cp,
        out_type=(
            jax.ShapeDtypeStruct((4 * NP, 64), jnp.float32),
            jax.ShapeDtypeStruct((2 * NCHUNK, C, 16), jnp.float32),
            jax.ShapeDtypeStruct((2 * NCHUNK, C, 16), jnp.float32),
        ),
        mesh=mesh,
        scratch_types=[
            pltpu.VMEM((2, C), jnp.int32),
            pltpu.VMEM((C,), jnp.int32),
            pltpu.VMEM((C, 16), jnp.float32),
            pltpu.VMEM((C, 16), jnp.float32),
            pltpu.VMEM((C, 16), jnp.float32),
            pltpu.VMEM((C, 16), jnp.float32),
            pltpu.VMEM((C, 64), jnp.float32),
            pltpu.VMEM((C,), jnp.int32),
            pltpu.VMEM_SHARED((NP, 16), jnp.float32),
            pltpu.VMEM_SHARED((NP, 64), jnp.float32),
        ],
    )
    def k(rc2_r, ti_r, tj_r, wh4_r, out_r, e_scr, ee_scr,
          rcv, adjv, buf_a, buf_b, buf_e, buf_s, rowsb, adjc,
          S12, OSP):
        c = lax.axis_index("c")
        s = lax.axis_index("s")

        zero16 = jnp.zeros((16,), jnp.float32)
        iota16 = lax.iota(jnp.int32, 16)
        mask48 = (iota16 >= 4) & (iota16 < 8)
        shidx = jnp.maximum(iota16 - 4, 0)
        nz = NP // NSUB

        def zero_rowsb():
            @plsc.parallel_loop(0, C, unroll=4)
            def _(r):
                for t in range(4):
                    rowsb[r, pl.ds(t * 16, 16)] = zero16

        def zero_osp():
            pltpu.sync_copy(rowsb, OSP.at[pl.ds(s * nz, C)])
            pltpu.sync_copy(rowsb.at[pl.ds(0, nz - C)],
                            OSP.at[pl.ds(s * nz + C, nz - C)])

        zero_rowsb()

        @plsc.parallel_loop(0, C, unroll=4)
        def _(r):
            buf_e[r, :] = zero16

        zero_osp()
        pltpu.sync_copy(buf_e, S12.at[pl.ds(s * nz, C)])
        pltpu.sync_copy(buf_e.at[pl.ds(0, nz - C)],
                        S12.at[pl.ds(s * nz + C, nz - C)])
        plsc.subcore_barrier()

        def adjust(plane, dst, base):
            @plsc.parallel_loop(0, C // 16, unroll=4)
            def _(qq):
                sl = pl.ds(qq * 16, 16)
                dst[sl] = rcv[plane, sl] + base

        @pl.loop(0, KPT)
        def _(kk):
            cid = s * KPT + kk
            gcid = c * NCHUNK + cid
            pltpu.sync_copy(rc2_r.at[cid], rcv)
            adjust(0, adjv, c * NP)
            adjust(1, adjc, c * NP)
            pltpu.sync_copy(ti_r.at[adjv], buf_a)
            pltpu.sync_copy(tj_r.at[adjc], buf_b)

            @plsc.parallel_loop(0, C, unroll=4)
            def _(r):
                v = buf_a[r, :] + buf_b[r, :]
                buf_e[r, :] = jnp.where(v > 0, v, v * ALPHA)

            pltpu.sync_copy(buf_e, e_scr.at[gcid])
            pltpu.sync_copy(buf_e, S12.at[rcv.at[0]], add=True)
        plsc.subcore_barrier()

        @pl.loop(0, KPT)
        def _(kk):
            cid = s * KPT + kk
            gcid = c * NCHUNK + cid
            pltpu.sync_copy(rc2_r.at[cid], rcv)
            pltpu.sync_copy(e_scr.at[gcid], buf_e)
            pltpu.sync_copy(S12.at[rcv.at[0]], buf_s)

            @plsc.parallel_loop(0, C, unroll=4)
            def _(r):
                ex = jnp.exp(buf_e[r, :] - buf_s[r, :])
                sh = ex.at[shidx].get(mode=lax.GatherScatterMode.PROMISE_IN_BOUNDS)
                buf_b[r, :] = jnp.where(mask48, sh, 0.0)

            pltpu.sync_copy(buf_b, ee_scr.at[gcid])
            pltpu.sync_copy(buf_b, S12.at[rcv.at[0]], add=True)
        plsc.subcore_barrier()

        for half in range(2):
            @pl.loop(0, KPT)
            def _(kk, half=half):
                cid = s * KPT + kk
                gcid = c * NCHUNK + cid
                pltpu.sync_copy(rc2_r.at[cid], rcv)
                pltpu.sync_copy(ee_scr.at[gcid], buf_b)
                if half == 0:
                    pltpu.sync_copy(S12.at[rcv.at[0]], buf_s)

                    @plsc.parallel_loop(0, C, unroll=4)
                    def _(r):
                        buf_b[r, :] = buf_b[r, :] / (buf_s[r, :] + 1e-16)

                    pltpu.sync_copy(buf_b, ee_scr.at[gcid])

                adjust(1, adjc, (c * 2 + half) * NP)
                pltpu.sync_copy(wh4_r.at[adjc], rowsb)

                @plsc.parallel_loop(0, C, unroll=2)
                def _(ce):
                    for hh in range(2):
                        av = plsc.load_gather(
                            buf_b,
                            [jnp.full((16,), ce, jnp.int32),
                             jnp.full((16,), 4 + half * 2 + hh, jnp.int32)])
                        for t in range(2):
                            sl = pl.ds(hh * 32 + t * 16, 16)
                            rowsb[ce, sl] = rowsb[ce, sl] * av
                pltpu.sync_copy(rowsb, OSP.at[rcv.at[0]], add=True)
            plsc.subcore_barrier()

            pltpu.sync_copy(
                OSP.at[pl.ds(s * nz, nz)],
                out_r.at[pl.ds((c * 2 + half) * NP + s * nz, nz)])
            if half == 0:
                plsc.subcore_barrier()
                zero_rowsb()
                zero_osp()
                plsc.subcore_barrier()

    return k(rc2, ti, tj, wh4)


def kernel(h, adj, W, a):
    h = h.astype(jnp.float32)
    W = W.astype(jnp.float32)
    a = a.astype(jnp.float32)

    h_pad = jnp.zeros((NP, IN_F), jnp.float32).at[:N].set(h)
    wc = (jnp.transpose(W, (1, 0, 2)).reshape(IN_F, 4, 64)
          .transpose(1, 0, 2))
    a1 = a[:, :OUT_F, 0].reshape(2, 4, OUT_F)
    a2 = a[:, OUT_F:, 0].reshape(2, 4, OUT_F)
    eye = jnp.eye(4, 16, dtype=jnp.float32)
    a1b = (a1[:, :, :, None] * eye[None, :, None, :]).reshape(2, 128, 16)
    a2b = (a2[:, :, :, None] * eye[None, :, None, :]).reshape(2, 128, 16)
    a1q = a1b.reshape(4, 64, 16)
    a2q = a2b.reshape(4, 64, 16)

    row = adj[0].astype(jnp.int32)
    col = adj[1].astype(jnp.int32)
    rowp = jnp.full((E_PAD,), DUMMY, jnp.int32).at[:E].set(row)
    colp = jnp.full((E_PAD,), DUMMY, jnp.int32).at[:E].set(col)
    rc2 = jnp.stack([rowp.reshape(NCHUNK, C),
                     colp.reshape(NCHUNK, C)], axis=1)

    wh4, ti, tj = _tc_proj(h_pad, wc, a1q, a2q)
    out4, _, _ = _sc_gat(rc2, ti, tj, wh4)
    return jnp.concatenate(
        [out4[q * NP:q * NP + N] for q in range(4)], axis=1)
